# Initial kernel scaffold; baseline (speedup 1.0000x reference)
import jax
import jax.numpy as jnp
from jax.experimental import pallas as pl


def kernel(node_table, edge_table, Wt, bt, We, be, Wa, ba, gamma, beta, node_features, edge_features, edge_index):
    raise NotImplementedError("write your pallas kernel here")


# speedup vs baseline: 21.4302x; 21.4302x over previous
import functools

import jax
import jax.numpy as jnp
from jax import lax
from jax.experimental import pallas as pl
from jax.experimental.pallas import tpu as pltpu
from jax.experimental.pallas import tpu_sc as plsc

N = 10000
E = 320000
D = 128
H = 4
F = 64
HF = 256
EDICT = 512

NT = 16
CH = 80
CH3 = 16
SB = 800
EPT = E // NT
EPT3 = E // (NT * 2)
DEN_PAD = 20480
NPAD = 10240


def _tc1_body(nt_ref, et_ref, wt_ref, bt_ref, we_ref, be_ref, w12_ref,
              w3_ref, ba_ref, g_ref, b_ref,
              nt2_ref, aiaj_ref, eta_ref, lnet_ref):
    ntt = jnp.dot(nt_ref[...], wt_ref[...],
                  preferred_element_type=jnp.float32) + bt_ref[...]
    nt2_ref[0:N, :] = ntt[:, 0:D]
    nt2_ref[N:2 * N, :] = ntt[:, D:HF]
    aiaj_ref[...] = jnp.dot(ntt, w12_ref[...],
                            preferred_element_type=jnp.float32)
    ett = jnp.dot(et_ref[...], we_ref[...],
                  preferred_element_type=jnp.float32) + be_ref[...]
    eta_ref[...] = jnp.dot(ett, w3_ref[...],
                           preferred_element_type=jnp.float32) + ba_ref[...]
    mu = jnp.mean(ett, axis=1, keepdims=True)
    var = jnp.mean((ett - mu) ** 2, axis=1, keepdims=True)
    lnet_ref[...] = (ett - mu) * lax.rsqrt(var + 1e-5) * g_ref[...] + b_ref[...]


def _tc1(node_table, edge_table, wt, bt2d, we, be2d, w12, w3, ba2d, g2d, b2d):
    return pl.pallas_call(
        _tc1_body,
        out_shape=[
            jax.ShapeDtypeStruct((2 * N, D), jnp.float32),
            jax.ShapeDtypeStruct((N, 8), jnp.float32),
            jax.ShapeDtypeStruct((EDICT, 8), jnp.float32),
            jax.ShapeDtypeStruct((EDICT, HF), jnp.float32),
        ],
    )(node_table, edge_table, wt, bt2d, we, be2d, w12, w3, ba2d, g2d, b2d)


def _tc2_body(acc_ref, xt_ref, deg_ref, g_ref, b_ref, out_ref):
    dg = deg_ref[...]
    y0 = acc_ref[0:N, :] + dg * xt_ref[0:N, :]
    y1 = acc_ref[N:2 * N, :] + dg * xt_ref[N:2 * N, :]
    y = jnp.concatenate([y0, y1], axis=1)
    mu = jnp.mean(y, axis=1, keepdims=True)
    var = jnp.mean((y - mu) ** 2, axis=1, keepdims=True)
    out_ref[...] = (y - mu) * lax.rsqrt(var + 1e-5) * g_ref[...] + b_ref[...]


def _tc2(acc, xt, deg2d, g2d, b2d):
    return pl.pallas_call(
        _tc2_body,
        out_shape=jax.ShapeDtypeStruct((N, HF), jnp.float32),
    )(acc, xt, deg2d, g2d, b2d)


def _sc_a(nt2, aij_cm, nf):
    mesh = plsc.VectorSubcoreMesh(core_axis_name="c", subcore_axis_name="s")

    @functools.partial(
        pl.kernel,
        mesh=mesh,
        compiler_params=pltpu.CompilerParams(needs_layout_passes=False),
        out_type=[
            jax.ShapeDtypeStruct((2 * N, D), jnp.float32),
            jax.ShapeDtypeStruct((8 * NPAD,), jnp.float32),
        ],
        scratch_types=[
            pltpu.VMEM((CH,), jnp.int32),
            pltpu.VMEM((CH,), jnp.int32),
            pltpu.VMEM((CH,), jnp.int32),
            pltpu.VMEM((CH,), jnp.int32),
            pltpu.VMEM((CH,), jnp.int32),
            pltpu.VMEM((CH,), jnp.int32),
            pltpu.VMEM((CH,), jnp.float32),
            pltpu.VMEM((CH,), jnp.float32),
            pltpu.VMEM((CH,), jnp.float32),
            pltpu.VMEM((CH,), jnp.float32),
            pltpu.VMEM((CH, D), jnp.float32),
            pltpu.SemaphoreType.DMA,
        ],
    )
    def k(nt2_h, aij_h, nf_h, xt_o, an_o,
          t1b, gia, gib, gja, gjb, rix_v, ai0b, ai1b, aj0b, aj1b, rows_v,
          sem):
        cid = lax.axis_index("c")
        tid = lax.axis_index("s")
        nbase = cid * N
        abase = cid * 4 * NPAD
        row_lo = tid * 640
        nch0 = jnp.where(tid < 15, 640 // CH, 400 // CH)

        def p0_chunk(c, carry):
            base = row_lo + c * CH
            pltpu.sync_copy(nf_h.at[pl.ds(base, CH)], t1b)
            for g in range(CH // 16):
                o = pl.ds(g * 16, 16)
                v = t1b[o]
                rix_v[o] = v + nbase
                gia[o] = v + abase
                gib[o] = v + (abase + NPAD)
                gja[o] = v + (abase + 2 * NPAD)
                gjb[o] = v + (abase + 3 * NPAD)
            cp_rw = pltpu.async_copy(nt2_h.at[rix_v], rows_v, sem)
            cp_a = pltpu.async_copy(aij_h.at[gia], ai0b, sem)
            cp_b = pltpu.async_copy(aij_h.at[gib], ai1b, sem)
            cp_c = pltpu.async_copy(aij_h.at[gja], aj0b, sem)
            cp_d = pltpu.async_copy(aij_h.at[gjb], aj1b, sem)
            cp_rw.wait()
            cp_a.wait()
            cp_b.wait()
            cp_c.wait()
            cp_d.wait()
            pltpu.sync_copy(rows_v, xt_o.at[pl.ds(nbase + base, CH)])
            pltpu.sync_copy(ai0b, an_o.at[pl.ds(abase + base, CH)])
            pltpu.sync_copy(ai1b, an_o.at[pl.ds(abase + NPAD + base, CH)])
            pltpu.sync_copy(aj0b, an_o.at[pl.ds(abase + 2 * NPAD + base, CH)])
            pltpu.sync_copy(aj1b, an_o.at[pl.ds(abase + 3 * NPAD + base, CH)])
            return carry

        lax.fori_loop(0, nch0, p0_chunk, 0)

    return k(nt2, aij_cm, nf)


def _sc_b(xt, an_cm, eta2, ln_et, ef, src, dst):
    mesh = plsc.VectorSubcoreMesh(core_axis_name="c", subcore_axis_name="s")

    @functools.partial(
        pl.kernel,
        mesh=mesh,
        compiler_params=pltpu.CompilerParams(needs_layout_passes=False),
        out_type=[
            jax.ShapeDtypeStruct((2 * N, D), jnp.float32),
            jax.ShapeDtypeStruct((NPAD,), jnp.float32),
            jax.ShapeDtypeStruct((E, HF), jnp.float32),
        ],
        scratch_types=[
            pltpu.VMEM((4 * EDICT,), jnp.float32),
            pltpu.VMEM((SB,), jnp.int32),
            pltpu.VMEM((SB,), jnp.int32),
            pltpu.VMEM((SB,), jnp.int32),
            pltpu.VMEM((CH,), jnp.int32),
            pltpu.VMEM((CH,), jnp.int32),
            pltpu.VMEM((CH,), jnp.int32),
            pltpu.VMEM((CH,), jnp.int32),
            pltpu.VMEM((CH,), jnp.float32),
            pltpu.VMEM((CH,), jnp.float32),
            pltpu.VMEM((CH,), jnp.float32),
            pltpu.VMEM((CH,), jnp.float32),
            pltpu.VMEM((CH,), jnp.float32),
            pltpu.VMEM((CH,), jnp.float32),
            pltpu.VMEM((CH,), jnp.float32),
            pltpu.VMEM((CH,), jnp.float32),
            pltpu.VMEM((CH,), jnp.int32),
            pltpu.VMEM((CH,), jnp.int32),
            pltpu.VMEM((CH,), jnp.int32),
            pltpu.VMEM((CH,), jnp.int32),
            pltpu.VMEM((CH,), jnp.float32),
            pltpu.VMEM((CH,), jnp.float32),
            pltpu.VMEM((CH,), jnp.float32),
            pltpu.VMEM((CH3,), jnp.int32),
            pltpu.VMEM((CH, D), jnp.float32),
            pltpu.VMEM((CH3, HF), jnp.float32),
            pltpu.VMEM_SHARED((N, D), jnp.float32),
            pltpu.VMEM_SHARED((DEN_PAD,), jnp.float32),
            pltpu.VMEM_SHARED((NPAD,), jnp.float32),
            pltpu.VMEM_SHARED((4 * NPAD,), jnp.float32),
            pltpu.SemaphoreType.DMA,
        ],
    )
    def k(xt_h, an_h, eta_h, lnet_h, ef_h, src_h, dst_h,
          acc_o, deg_o, eout_o,
          eta_v, s_big, d_big, f_big, gia, gib, gja, gjb,
          ai0b, ai1b, aj0b, aj1b, dn0b, dn1b,
          ex0_v, ex1_v, di0_v, di1_v, dd_v, rix_v, at0_v, at1_v, ones_v,
          fch_v, rows_v, erows_v, accum_sp, den_sp, deg_sp, an_sp, sem):
        cid = lax.axis_index("c")
        tid = lax.axis_index("s")
        nbase = cid * N
        abase = cid * 4 * NPAD
        eb0 = cid * 2 * EDICT

        for g in range(CH // 16):
            ones_v[pl.ds(g * 16, 16)] = jnp.ones((16,), jnp.float32)
            ex0_v[pl.ds(g * 16, 16)] = jnp.zeros((16,), jnp.float32)
        row_lo = tid * 640
        nch0 = jnp.where(tid < 15, 640 // CH, 400 // CH)

        def zrow(r, carry):
            for q in range(D // 16):
                rows_v[r, pl.ds(q * 16, 16)] = jnp.zeros((16,), jnp.float32)
            return carry

        pltpu.sync_copy(eta_h, eta_v)
        for i in range(4 * NPAD // NT // CH):
            lo = tid * (4 * NPAD // NT) + i * CH
            pltpu.sync_copy(an_h.at[pl.ds(abase + lo, CH)], ai0b)
            pltpu.sync_copy(ai0b, an_sp.at[pl.ds(lo, CH)])
        for i in range(DEN_PAD // NT // CH):
            pltpu.sync_copy(
                ex0_v, den_sp.at[pl.ds(tid * (DEN_PAD // NT) + i * CH, CH)])
        for i in range(NPAD // NT // CH):
            pltpu.sync_copy(
                ex0_v, deg_sp.at[pl.ds(tid * (NPAD // NT) + i * CH, CH)])
        lax.fori_loop(0, CH, zrow, 0)

        def zacc(c, carry):
            pltpu.sync_copy(rows_v, accum_sp.at[pl.ds(row_lo + c * CH, CH)])
            return carry

        lax.fori_loop(0, nch0, zacc, 0)

        plsc.subcore_barrier()

        def p1_super(u, carry):
            eb = tid * EPT + u * SB
            pltpu.sync_copy(src_h.at[pl.ds(eb, SB)], s_big)
            pltpu.sync_copy(dst_h.at[pl.ds(eb, SB)], d_big)
            pltpu.sync_copy(ef_h.at[pl.ds(eb, SB)], f_big)

            def p1_chunk(v, c2):
                off = v * CH
                for g in range(CH // 16):
                    o = pl.ds(g * 16, 16)
                    s = s_big[pl.ds(off + g * 16, 16)]
                    d = d_big[pl.ds(off + g * 16, 16)]
                    gia[o] = d
                    gib[o] = d + NPAD
                    gja[o] = s + 2 * NPAD
                    gjb[o] = s + 3 * NPAD
                    di0_v[o] = d * 2
                    di1_v[o] = d * 2 + 1
                    dd_v[o] = d
                cp_a = pltpu.async_copy(an_sp.at[gia], ai0b, sem)
                cp_b = pltpu.async_copy(an_sp.at[gib], ai1b, sem)
                cp_c = pltpu.async_copy(an_sp.at[gja], aj0b, sem)
                cp_d = pltpu.async_copy(an_sp.at[gjb], aj1b, sem)
                cp_a.wait()
                cp_b.wait()
                cp_c.wait()
                cp_d.wait()
                for g in range(CH // 16):
                    o = pl.ds(g * 16, 16)
                    f = f_big[pl.ds(off + g * 16, 16)]
                    a_e0 = plsc.load_gather(eta_v, [eb0 + f * 2])
                    a_e1 = plsc.load_gather(eta_v, [eb0 + f * 2 + 1])
                    l0 = ai0b[o] + aj0b[o] + a_e0
                    l1 = ai1b[o] + aj1b[o] + a_e1
                    l0 = jnp.where(l0 >= 0.0, l0, 0.2 * l0)
                    l1 = jnp.where(l1 >= 0.0, l1, 0.2 * l1)
                    ex0_v[o] = jnp.exp(l0)
                    ex1_v[o] = jnp.exp(l1)
                pltpu.sync_copy(ex0_v, den_sp.at[di0_v], add=True)
                pltpu.sync_copy(ex1_v, den_sp.at[di1_v], add=True)

                @pl.when(cid == 0)
                def _():
                    pltpu.sync_copy(ones_v, deg_sp.at[dd_v], add=True)
                return c2

            lax.fori_loop(0, SB // CH, p1_chunk, 0)
            return carry

        lax.fori_loop(0, EPT // SB, p1_super, 0)
        plsc.subcore_barrier()

        def p2_super(u, carry):
            eb = tid * EPT + u * SB
            pltpu.sync_copy(src_h.at[pl.ds(eb, SB)], s_big)
            pltpu.sync_copy(dst_h.at[pl.ds(eb, SB)], d_big)
            pltpu.sync_copy(ef_h.at[pl.ds(eb, SB)], f_big)

            def p2_chunk(v, c2):
                off = v * CH
                for g in range(CH // 16):
                    o = pl.ds(g * 16, 16)
                    s = s_big[pl.ds(off + g * 16, 16)]
                    d = d_big[pl.ds(off + g * 16, 16)]
                    gia[o] = d
                    gib[o] = d + NPAD
                    gja[o] = s + 2 * NPAD
                    gjb[o] = s + 3 * NPAD
                    rix_v[o] = s + nbase
                    di0_v[o] = d * 2
                    di1_v[o] = d * 2 + 1
                    dd_v[o] = d
                cp_a = pltpu.async_copy(an_sp.at[gia], ai0b, sem)
                cp_b = pltpu.async_copy(an_sp.at[gib], ai1b, sem)
                cp_c = pltpu.async_copy(an_sp.at[gja], aj0b, sem)
                cp_d = pltpu.async_copy(an_sp.at[gjb], aj1b, sem)
                cp_d0 = pltpu.async_copy(den_sp.at[di0_v], dn0b, sem)
                cp_d1 = pltpu.async_copy(den_sp.at[di1_v], dn1b, sem)
                cp_a.wait()
                cp_b.wait()
                cp_c.wait()
                cp_d.wait()
                cp_d0.wait()
                cp_d1.wait()
                pltpu.async_copy(xt_h.at[rix_v], rows_v, sem).wait()
                for g in range(CH // 16):
                    o = pl.ds(g * 16, 16)
                    f = f_big[pl.ds(off + g * 16, 16)]
                    a_e0 = plsc.load_gather(eta_v, [eb0 + f * 2])
                    a_e1 = plsc.load_gather(eta_v, [eb0 + f * 2 + 1])
                    l0 = ai0b[o] + aj0b[o] + a_e0
                    l1 = ai1b[o] + aj1b[o] + a_e1
                    l0 = jnp.where(l0 >= 0.0, l0, 0.2 * l0)
                    l1 = jnp.where(l1 >= 0.0, l1, 0.2 * l1)
                    at0_v[o] = jnp.exp(l0) / dn0b[o]
                    at1_v[o] = jnp.exp(l1) / dn1b[o]
                for g in range(CH // 16):
                    a0v = at0_v[pl.ds(g * 16, 16)]
                    a1v = at1_v[pl.ds(g * 16, 16)]
                    for b in range(16):
                        r = g * 16 + b
                        ib = jnp.full((16,), b, jnp.int32)
                        c0 = a0v.at[ib].get(mode="promise_in_bounds")
                        c1 = a1v.at[ib].get(mode="promise_in_bounds")
                        for q in range(4):
                            oq = pl.ds(q * 16, 16)
                            rows_v[r, oq] = rows_v[r, oq] * c0
                        for q in range(4, 8):
                            oq = pl.ds(q * 16, 16)
                            rows_v[r, oq] = rows_v[r, oq] * c1
                pltpu.sync_copy(rows_v, accum_sp.at[dd_v], add=True)
                return c2

            lax.fori_loop(0, SB // CH, p2_chunk, 0)
            return carry

        lax.fori_loop(0, EPT // SB, p2_super, 0)
        plsc.subcore_barrier()

        def wacc(c, carry):
            lo = row_lo + c * CH
            pltpu.sync_copy(accum_sp.at[pl.ds(lo, CH)], rows_v)
            pltpu.sync_copy(rows_v, acc_o.at[pl.ds(nbase + lo, CH)])
            return carry

        lax.fori_loop(0, nch0, wacc, 0)

        @pl.when(cid == 0)
        def _():
            for i in range(NPAD // NT // CH):
                lo = tid * (NPAD // NT) + i * CH
                pltpu.sync_copy(deg_sp.at[pl.ds(lo, CH)], ex0_v)
                pltpu.sync_copy(ex0_v, deg_o.at[pl.ds(lo, CH)])

        def p3_chunk(c, carry):
            base = cid * (E // 2) + tid * EPT3 + c * CH3
            pltpu.sync_copy(ef_h.at[pl.ds(base, CH3)], fch_v)
            pltpu.async_copy(lnet_h.at[fch_v], erows_v, sem).wait()
            pltpu.sync_copy(erows_v, eout_o.at[pl.ds(base, CH3)])
            return carry

        lax.fori_loop(0, EPT3 // CH3, p3_chunk, 0)

    return k(xt, an_cm, eta2, ln_et, ef, src, dst)


def kernel(node_table, edge_table, Wt, bt, We, be, Wa, ba, gamma, beta,
           node_features, edge_features, edge_index):
    f32 = jnp.float32
    nf = node_features.astype(jnp.int32)
    ef = edge_features.astype(jnp.int32)
    src = edge_index[0].astype(jnp.int32)
    dst = edge_index[1].astype(jnp.int32)
    wa1 = Wa[0:F, 0]
    wa2 = Wa[F:2 * F, 0]
    wa3 = Wa[2 * F:3 * F, 0]
    eye = jnp.eye(H, dtype=f32)
    w1 = jnp.kron(eye, wa1[:, None])
    w2 = jnp.kron(eye, wa2[:, None])
    w12 = jnp.concatenate([w1, w2], axis=1)
    w3 = jnp.concatenate([jnp.kron(eye, wa3[:, None]),
                          jnp.zeros((HF, H), f32)], axis=1)
    nt2, aiaj, eta8, ln_et = _tc1(
        node_table.astype(f32), edge_table.astype(f32), Wt.astype(f32),
        bt.reshape(1, HF).astype(f32), We.astype(f32),
        be.reshape(1, HF).astype(f32), w12, w3,
        ba.reshape(1, 1).astype(f32), gamma.reshape(1, HF).astype(f32),
        beta.reshape(1, HF).astype(f32))
    ai = aiaj[:, 0:H]
    aj = aiaj[:, H:2 * H]
    et_a = eta8[:, 0:H]
    pad = jnp.zeros((NPAD - N,), f32)
    cols = []
    for core in range(2):
        for col in (ai[:, 2 * core], ai[:, 2 * core + 1],
                    aj[:, 2 * core], aj[:, 2 * core + 1]):
            cols.append(jnp.concatenate([col, pad]))
    aij_cm = jnp.concatenate(cols)
    eta2 = jnp.concatenate([et_a[:, 0:2].reshape(-1),
                            et_a[:, 2:4].reshape(-1)])
    xt, an_cm = _sc_a(nt2, aij_cm, nf)
    acc, deg, edge_out = _sc_b(xt, an_cm, eta2, ln_et, ef, src, dst)
    deg2d = deg[0:N].reshape(N, 1)
    out = _tc2(acc, xt, deg2d, gamma.reshape(1, HF).astype(f32),
               beta.reshape(1, HF).astype(f32))
    return (out, edge_out)


# speedup vs baseline: 29.5151x; 1.3773x over previous
import functools

import jax
import jax.numpy as jnp
from jax import lax
from jax.experimental import pallas as pl
from jax.experimental.pallas import tpu as pltpu
from jax.experimental.pallas import tpu_sc as plsc

N = 10000
E = 320000
D = 128
H = 4
F = 64
HF = 256
EDICT = 512

NT = 16
CH = 80
CH3 = 80
SB = 800
EPT = E // NT
EPT3 = E // (NT * 2)
DEN_PAD = 20480
NPAD = 10240


def _tc1_body(nt_ref, et_ref, wt_ref, bt_ref, we_ref, be_ref, w12_ref,
              w3_ref, ba_ref, g_ref, b_ref,
              nt2_ref, aiaj_ref, eta_ref, lnet_ref):
    ntt = jnp.dot(nt_ref[...], wt_ref[...],
                  preferred_element_type=jnp.float32) + bt_ref[...]
    nt2_ref[0:N, :] = ntt[:, 0:D]
    nt2_ref[N:2 * N, :] = ntt[:, D:HF]
    aiaj_ref[...] = jnp.dot(ntt, w12_ref[...],
                            preferred_element_type=jnp.float32)
    ett = jnp.dot(et_ref[...], we_ref[...],
                  preferred_element_type=jnp.float32) + be_ref[...]
    eta_ref[...] = jnp.dot(ett, w3_ref[...],
                           preferred_element_type=jnp.float32) + ba_ref[...]
    mu = jnp.mean(ett, axis=1, keepdims=True)
    var = jnp.mean((ett - mu) ** 2, axis=1, keepdims=True)
    lnet_ref[...] = (ett - mu) * lax.rsqrt(var + 1e-5) * g_ref[...] + b_ref[...]


def _tc1(node_table, edge_table, wt, bt2d, we, be2d, w12, w3, ba2d, g2d, b2d):
    return pl.pallas_call(
        _tc1_body,
        out_shape=[
            jax.ShapeDtypeStruct((2 * N, D), jnp.float32),
            jax.ShapeDtypeStruct((N, 8), jnp.float32),
            jax.ShapeDtypeStruct((EDICT, 8), jnp.float32),
            jax.ShapeDtypeStruct((EDICT, HF), jnp.float32),
        ],
    )(node_table, edge_table, wt, bt2d, we, be2d, w12, w3, ba2d, g2d, b2d)


def _tc2_body(acc_ref, xt_ref, deg_ref, g_ref, b_ref, out_ref):
    dg = deg_ref[...]
    y0 = acc_ref[0:N, :] + dg * xt_ref[0:N, :]
    y1 = acc_ref[N:2 * N, :] + dg * xt_ref[N:2 * N, :]
    y = jnp.concatenate([y0, y1], axis=1)
    mu = jnp.mean(y, axis=1, keepdims=True)
    var = jnp.mean((y - mu) ** 2, axis=1, keepdims=True)
    out_ref[...] = (y - mu) * lax.rsqrt(var + 1e-5) * g_ref[...] + b_ref[...]


def _tc2(acc, xt, deg2d, g2d, b2d):
    return pl.pallas_call(
        _tc2_body,
        out_shape=jax.ShapeDtypeStruct((N, HF), jnp.float32),
    )(acc, xt, deg2d, g2d, b2d)


def _sc_a(nt2, aij_cm, nf):
    mesh = plsc.VectorSubcoreMesh(core_axis_name="c", subcore_axis_name="s")

    @functools.partial(
        pl.kernel,
        mesh=mesh,
        compiler_params=pltpu.CompilerParams(needs_layout_passes=False),
        out_type=[
            jax.ShapeDtypeStruct((2 * N, D), jnp.float32),
            jax.ShapeDtypeStruct((8 * NPAD,), jnp.float32),
        ],
        scratch_types=[
            pltpu.VMEM((CH,), jnp.int32),
            pltpu.VMEM((CH,), jnp.int32),
            pltpu.VMEM((CH,), jnp.int32),
            pltpu.VMEM((CH,), jnp.int32),
            pltpu.VMEM((CH,), jnp.int32),
            pltpu.VMEM((CH,), jnp.int32),
            pltpu.VMEM((CH,), jnp.float32),
            pltpu.VMEM((CH,), jnp.float32),
            pltpu.VMEM((CH,), jnp.float32),
            pltpu.VMEM((CH,), jnp.float32),
            pltpu.VMEM((CH, D), jnp.float32),
            pltpu.SemaphoreType.DMA,
        ],
    )
    def k(nt2_h, aij_h, nf_h, xt_o, an_o,
          t1b, gia, gib, gja, gjb, rix_v, ai0b, ai1b, aj0b, aj1b, rows_v,
          sem):
        cid = lax.axis_index("c")
        tid = lax.axis_index("s")
        nbase = cid * N
        abase = cid * 4 * NPAD
        row_lo = tid * 640
        nch0 = jnp.where(tid < 15, 640 // CH, 400 // CH)

        def p0_chunk(c, carry):
            base = row_lo + c * CH
            pltpu.sync_copy(nf_h.at[pl.ds(base, CH)], t1b)
            for g in range(CH // 16):
                o = pl.ds(g * 16, 16)
                v = t1b[o]
                rix_v[o] = v + nbase
                gia[o] = v + abase
                gib[o] = v + (abase + NPAD)
                gja[o] = v + (abase + 2 * NPAD)
                gjb[o] = v + (abase + 3 * NPAD)
            cp_rw = pltpu.async_copy(nt2_h.at[rix_v], rows_v, sem)
            cp_a = pltpu.async_copy(aij_h.at[gia], ai0b, sem)
            cp_b = pltpu.async_copy(aij_h.at[gib], ai1b, sem)
            cp_c = pltpu.async_copy(aij_h.at[gja], aj0b, sem)
            cp_d = pltpu.async_copy(aij_h.at[gjb], aj1b, sem)
            cp_rw.wait()
            cp_a.wait()
            cp_b.wait()
            cp_c.wait()
            cp_d.wait()
            pltpu.sync_copy(rows_v, xt_o.at[pl.ds(nbase + base, CH)])
            pltpu.sync_copy(ai0b, an_o.at[pl.ds(abase + base, CH)])
            pltpu.sync_copy(ai1b, an_o.at[pl.ds(abase + NPAD + base, CH)])
            pltpu.sync_copy(aj0b, an_o.at[pl.ds(abase + 2 * NPAD + base, CH)])
            pltpu.sync_copy(aj1b, an_o.at[pl.ds(abase + 3 * NPAD + base, CH)])
            return carry

        lax.fori_loop(0, nch0, p0_chunk, 0)

    return k(nt2, aij_cm, nf)


def _sc_b(xt, an_cm, eta2, ln_et, ef, src, dst):
    mesh = plsc.VectorSubcoreMesh(core_axis_name="c", subcore_axis_name="s")

    @functools.partial(
        pl.kernel,
        mesh=mesh,
        compiler_params=pltpu.CompilerParams(needs_layout_passes=False),
        out_type=[
            jax.ShapeDtypeStruct((2 * N, D), jnp.float32),
            jax.ShapeDtypeStruct((NPAD,), jnp.float32),
            jax.ShapeDtypeStruct((E, HF), jnp.float32),
        ],
        scratch_types=[
            pltpu.VMEM((4 * EDICT,), jnp.float32),
            pltpu.VMEM((SB,), jnp.int32),
            pltpu.VMEM((SB,), jnp.int32),
            pltpu.VMEM((SB,), jnp.int32),
            pltpu.VMEM((CH,), jnp.int32),
            pltpu.VMEM((CH,), jnp.int32),
            pltpu.VMEM((CH,), jnp.int32),
            pltpu.VMEM((CH,), jnp.int32),
            pltpu.VMEM((CH,), jnp.float32),
            pltpu.VMEM((CH,), jnp.float32),
            pltpu.VMEM((CH,), jnp.float32),
            pltpu.VMEM((CH,), jnp.float32),
            pltpu.VMEM((CH,), jnp.float32),
            pltpu.VMEM((CH,), jnp.float32),
            pltpu.VMEM((CH,), jnp.float32),
            pltpu.VMEM((CH,), jnp.float32),
            pltpu.VMEM((CH,), jnp.int32),
            pltpu.VMEM((CH,), jnp.int32),
            pltpu.VMEM((CH,), jnp.int32),
            pltpu.VMEM((CH,), jnp.int32),
            pltpu.VMEM((CH,), jnp.float32),
            pltpu.VMEM((CH,), jnp.float32),
            pltpu.VMEM((CH,), jnp.float32),
            pltpu.VMEM((CH3,), jnp.int32),
            pltpu.VMEM((CH, D), jnp.float32),
            pltpu.VMEM((CH3, HF), jnp.float32),
            pltpu.VMEM_SHARED((N, D), jnp.float32),
            pltpu.VMEM_SHARED((DEN_PAD,), jnp.float32),
            pltpu.VMEM_SHARED((NPAD,), jnp.float32),
            pltpu.VMEM_SHARED((4 * NPAD,), jnp.float32),
            pltpu.SemaphoreType.DMA,
        ],
    )
    def k(xt_h, an_h, eta_h, lnet_h, ef_h, src_h, dst_h,
          acc_o, deg_o, eout_o,
          eta_v, s_big, d_big, f_big, gia, gib, gja, gjb,
          ai0b, ai1b, aj0b, aj1b, dn0b, dn1b,
          ex0_v, ex1_v, di0_v, di1_v, dd_v, rix_v, at0_v, at1_v, ones_v,
          fch_v, rows_v, erows_v, accum_sp, den_sp, deg_sp, an_sp, sem):
        cid = lax.axis_index("c")
        tid = lax.axis_index("s")
        nbase = cid * N
        abase = cid * 4 * NPAD
        eb0 = cid * 2 * EDICT

        for g in range(CH // 16):
            ones_v[pl.ds(g * 16, 16)] = jnp.ones((16,), jnp.float32)
            ex0_v[pl.ds(g * 16, 16)] = jnp.zeros((16,), jnp.float32)
        row_lo = tid * 640
        nch0 = jnp.where(tid < 15, 640 // CH, 400 // CH)

        def zrow(r, carry):
            for q in range(D // 16):
                rows_v[r, pl.ds(q * 16, 16)] = jnp.zeros((16,), jnp.float32)
            return carry

        pltpu.sync_copy(eta_h, eta_v)
        for i in range(4 * NPAD // NT // CH):
            lo = tid * (4 * NPAD // NT) + i * CH
            pltpu.sync_copy(an_h.at[pl.ds(abase + lo, CH)], ai0b)
            pltpu.sync_copy(ai0b, an_sp.at[pl.ds(lo, CH)])
        for i in range(DEN_PAD // NT // CH):
            pltpu.sync_copy(
                ex0_v, den_sp.at[pl.ds(tid * (DEN_PAD // NT) + i * CH, CH)])
        for i in range(NPAD // NT // CH):
            pltpu.sync_copy(
                ex0_v, deg_sp.at[pl.ds(tid * (NPAD // NT) + i * CH, CH)])
        lax.fori_loop(0, CH, zrow, 0)

        def zacc(c, carry):
            pltpu.sync_copy(rows_v, accum_sp.at[pl.ds(row_lo + c * CH, CH)])
            return carry

        lax.fori_loop(0, nch0, zacc, 0)

        plsc.subcore_barrier()

        def p1_super(u, carry):
            eb = tid * EPT + u * SB
            pltpu.sync_copy(src_h.at[pl.ds(eb, SB)], s_big)
            pltpu.sync_copy(dst_h.at[pl.ds(eb, SB)], d_big)
            pltpu.sync_copy(ef_h.at[pl.ds(eb, SB)], f_big)

            def p1_chunk(v, c2):
                off = v * CH
                for g in range(CH // 16):
                    o = pl.ds(g * 16, 16)
                    s = s_big[pl.ds(off + g * 16, 16)]
                    d = d_big[pl.ds(off + g * 16, 16)]
                    gia[o] = d
                    gib[o] = d + NPAD
                    gja[o] = s + 2 * NPAD
                    gjb[o] = s + 3 * NPAD
                    di0_v[o] = d * 2
                    di1_v[o] = d * 2 + 1
                    dd_v[o] = d
                cp_a = pltpu.async_copy(an_sp.at[gia], ai0b, sem)
                cp_b = pltpu.async_copy(an_sp.at[gib], ai1b, sem)
                cp_c = pltpu.async_copy(an_sp.at[gja], aj0b, sem)
                cp_d = pltpu.async_copy(an_sp.at[gjb], aj1b, sem)
                cp_a.wait()
                cp_b.wait()
                cp_c.wait()
                cp_d.wait()
                for g in range(CH // 16):
                    o = pl.ds(g * 16, 16)
                    f = f_big[pl.ds(off + g * 16, 16)]
                    a_e0 = plsc.load_gather(eta_v, [eb0 + f * 2])
                    a_e1 = plsc.load_gather(eta_v, [eb0 + f * 2 + 1])
                    l0 = ai0b[o] + aj0b[o] + a_e0
                    l1 = ai1b[o] + aj1b[o] + a_e1
                    l0 = jnp.where(l0 >= 0.0, l0, 0.2 * l0)
                    l1 = jnp.where(l1 >= 0.0, l1, 0.2 * l1)
                    ex0_v[o] = jnp.exp(l0)
                    ex1_v[o] = jnp.exp(l1)
                pltpu.sync_copy(ex0_v, den_sp.at[di0_v], add=True)
                pltpu.sync_copy(ex1_v, den_sp.at[di1_v], add=True)

                @pl.when(cid == 0)
                def _():
                    pltpu.sync_copy(ones_v, deg_sp.at[dd_v], add=True)
                return c2

            lax.fori_loop(0, SB // CH, p1_chunk, 0)
            return carry

        lax.fori_loop(0, EPT // SB, p1_super, 0)
        plsc.subcore_barrier()

        def p2_super(u, carry):
            eb = tid * EPT + u * SB
            pltpu.sync_copy(src_h.at[pl.ds(eb, SB)], s_big)
            pltpu.sync_copy(dst_h.at[pl.ds(eb, SB)], d_big)
            pltpu.sync_copy(ef_h.at[pl.ds(eb, SB)], f_big)

            def p2_chunk(v, c2):
                off = v * CH
                for g in range(CH // 16):
                    o = pl.ds(g * 16, 16)
                    s = s_big[pl.ds(off + g * 16, 16)]
                    d = d_big[pl.ds(off + g * 16, 16)]
                    gia[o] = d
                    gib[o] = d + NPAD
                    gja[o] = s + 2 * NPAD
                    gjb[o] = s + 3 * NPAD
                    rix_v[o] = s + nbase
                    di0_v[o] = d * 2
                    di1_v[o] = d * 2 + 1
                    dd_v[o] = d
                cp_a = pltpu.async_copy(an_sp.at[gia], ai0b, sem)
                cp_b = pltpu.async_copy(an_sp.at[gib], ai1b, sem)
                cp_c = pltpu.async_copy(an_sp.at[gja], aj0b, sem)
                cp_d = pltpu.async_copy(an_sp.at[gjb], aj1b, sem)
                cp_d0 = pltpu.async_copy(den_sp.at[di0_v], dn0b, sem)
                cp_d1 = pltpu.async_copy(den_sp.at[di1_v], dn1b, sem)
                cp_a.wait()
                cp_b.wait()
                cp_c.wait()
                cp_d.wait()
                cp_d0.wait()
                cp_d1.wait()
                pltpu.async_copy(xt_h.at[rix_v], rows_v, sem).wait()
                for g in range(CH // 16):
                    o = pl.ds(g * 16, 16)
                    f = f_big[pl.ds(off + g * 16, 16)]
                    a_e0 = plsc.load_gather(eta_v, [eb0 + f * 2])
                    a_e1 = plsc.load_gather(eta_v, [eb0 + f * 2 + 1])
                    l0 = ai0b[o] + aj0b[o] + a_e0
                    l1 = ai1b[o] + aj1b[o] + a_e1
                    l0 = jnp.where(l0 >= 0.0, l0, 0.2 * l0)
                    l1 = jnp.where(l1 >= 0.0, l1, 0.2 * l1)
                    at0_v[o] = jnp.exp(l0) / dn0b[o]
                    at1_v[o] = jnp.exp(l1) / dn1b[o]
                for g in range(CH // 16):
                    a0v = at0_v[pl.ds(g * 16, 16)]
                    a1v = at1_v[pl.ds(g * 16, 16)]
                    for b in range(16):
                        r = g * 16 + b
                        ib = jnp.full((16,), b, jnp.int32)
                        c0 = a0v.at[ib].get(mode="promise_in_bounds")
                        c1 = a1v.at[ib].get(mode="promise_in_bounds")
                        for q in range(4):
                            oq = pl.ds(q * 16, 16)
                            rows_v[r, oq] = rows_v[r, oq] * c0
                        for q in range(4, 8):
                            oq = pl.ds(q * 16, 16)
                            rows_v[r, oq] = rows_v[r, oq] * c1
                pltpu.sync_copy(rows_v, accum_sp.at[dd_v], add=True)
                return c2

            lax.fori_loop(0, SB // CH, p2_chunk, 0)
            return carry

        lax.fori_loop(0, EPT // SB, p2_super, 0)
        plsc.subcore_barrier()

        def wacc(c, carry):
            lo = row_lo + c * CH
            pltpu.sync_copy(accum_sp.at[pl.ds(lo, CH)], rows_v)
            pltpu.sync_copy(rows_v, acc_o.at[pl.ds(nbase + lo, CH)])
            return carry

        lax.fori_loop(0, nch0, wacc, 0)

        @pl.when(cid == 0)
        def _():
            for i in range(NPAD // NT // CH):
                lo = tid * (NPAD // NT) + i * CH
                pltpu.sync_copy(deg_sp.at[pl.ds(lo, CH)], ex0_v)
                pltpu.sync_copy(ex0_v, deg_o.at[pl.ds(lo, CH)])

        def p3_chunk(c, carry):
            base = cid * (E // 2) + tid * EPT3 + c * CH3
            pltpu.sync_copy(ef_h.at[pl.ds(base, CH3)], fch_v)
            pltpu.async_copy(lnet_h.at[fch_v], erows_v, sem).wait()
            pltpu.sync_copy(erows_v, eout_o.at[pl.ds(base, CH3)])
            return carry

        lax.fori_loop(0, EPT3 // CH3, p3_chunk, 0)

    return k(xt, an_cm, eta2, ln_et, ef, src, dst)


def kernel(node_table, edge_table, Wt, bt, We, be, Wa, ba, gamma, beta,
           node_features, edge_features, edge_index):
    f32 = jnp.float32
    nf = node_features.astype(jnp.int32)
    ef = edge_features.astype(jnp.int32)
    src = edge_index[0].astype(jnp.int32)
    dst = edge_index[1].astype(jnp.int32)
    wa1 = Wa[0:F, 0]
    wa2 = Wa[F:2 * F, 0]
    wa3 = Wa[2 * F:3 * F, 0]
    eye = jnp.eye(H, dtype=f32)
    w1 = jnp.kron(eye, wa1[:, None])
    w2 = jnp.kron(eye, wa2[:, None])
    w12 = jnp.concatenate([w1, w2], axis=1)
    w3 = jnp.concatenate([jnp.kron(eye, wa3[:, None]),
                          jnp.zeros((HF, H), f32)], axis=1)
    nt2, aiaj, eta8, ln_et = _tc1(
        node_table.astype(f32), edge_table.astype(f32), Wt.astype(f32),
        bt.reshape(1, HF).astype(f32), We.astype(f32),
        be.reshape(1, HF).astype(f32), w12, w3,
        ba.reshape(1, 1).astype(f32), gamma.reshape(1, HF).astype(f32),
        beta.reshape(1, HF).astype(f32))
    ai = aiaj[:, 0:H]
    aj = aiaj[:, H:2 * H]
    et_a = eta8[:, 0:H]
    pad = jnp.zeros((NPAD - N,), f32)
    cols = []
    for core in range(2):
        for col in (ai[:, 2 * core], ai[:, 2 * core + 1],
                    aj[:, 2 * core], aj[:, 2 * core + 1]):
            cols.append(jnp.concatenate([col, pad]))
    aij_cm = jnp.concatenate(cols)
    eta2 = jnp.concatenate([et_a[:, 0:2].reshape(-1),
                            et_a[:, 2:4].reshape(-1)])
    xt, an_cm = _sc_a(nt2, aij_cm, nf)
    acc, deg, edge_out = _sc_b(xt, an_cm, eta2, ln_et, ef, src, dst)
    deg2d = deg[0:N].reshape(N, 1)
    out = _tc2(acc, xt, deg2d, gamma.reshape(1, HF).astype(f32),
               beta.reshape(1, HF).astype(f32))
    return (out, edge_out)


# speedup vs baseline: 32.4970x; 1.1010x over previous
import functools

import jax
import jax.numpy as jnp
from jax import lax
from jax.experimental import pallas as pl
from jax.experimental.pallas import tpu as pltpu
from jax.experimental.pallas import tpu_sc as plsc

N = 10000
E = 320000
D = 128
H = 4
F = 64
HF = 256
EDICT = 512

NT = 16
CH = 80
CH3 = 80
SB = 800
EPT = E // NT
EPT3 = E // (NT * 2)
DEN_PAD = 20480
NPAD = 10240


def _tc1_body(nt_ref, et_ref, wt_ref, bt_ref, we_ref, be_ref, w12_ref,
              w3_ref, ba_ref, g_ref, b_ref,
              nt2_ref, aiaj_ref, eta_ref, lnet_ref):
    ntt = jnp.dot(nt_ref[...], wt_ref[...],
                  preferred_element_type=jnp.float32) + bt_ref[...]
    nt2_ref[0:N, :] = ntt[:, 0:D]
    nt2_ref[N:2 * N, :] = ntt[:, D:HF]
    aiaj_ref[...] = jnp.dot(ntt, w12_ref[...],
                            preferred_element_type=jnp.float32)
    ett = jnp.dot(et_ref[...], we_ref[...],
                  preferred_element_type=jnp.float32) + be_ref[...]
    eta_ref[...] = jnp.dot(ett, w3_ref[...],
                           preferred_element_type=jnp.float32) + ba_ref[...]
    mu = jnp.mean(ett, axis=1, keepdims=True)
    var = jnp.mean((ett - mu) ** 2, axis=1, keepdims=True)
    lnet_ref[...] = (ett - mu) * lax.rsqrt(var + 1e-5) * g_ref[...] + b_ref[...]


def _tc1(node_table, edge_table, wt, bt2d, we, be2d, w12, w3, ba2d, g2d, b2d):
    return pl.pallas_call(
        _tc1_body,
        out_shape=[
            jax.ShapeDtypeStruct((2 * N, D), jnp.float32),
            jax.ShapeDtypeStruct((N, 8), jnp.float32),
            jax.ShapeDtypeStruct((EDICT, 8), jnp.float32),
            jax.ShapeDtypeStruct((EDICT, HF), jnp.float32),
        ],
    )(node_table, edge_table, wt, bt2d, we, be2d, w12, w3, ba2d, g2d, b2d)


def _tc2_body(acc_ref, xt_ref, deg_ref, g_ref, b_ref, out_ref):
    dg = deg_ref[...]
    y0 = acc_ref[0:N, :] + dg * xt_ref[0:N, :]
    y1 = acc_ref[N:2 * N, :] + dg * xt_ref[N:2 * N, :]
    y = jnp.concatenate([y0, y1], axis=1)
    mu = jnp.mean(y, axis=1, keepdims=True)
    var = jnp.mean((y - mu) ** 2, axis=1, keepdims=True)
    out_ref[...] = (y - mu) * lax.rsqrt(var + 1e-5) * g_ref[...] + b_ref[...]


def _tc2(acc, xt, deg2d, g2d, b2d):
    return pl.pallas_call(
        _tc2_body,
        out_shape=jax.ShapeDtypeStruct((N, HF), jnp.float32),
    )(acc, xt, deg2d, g2d, b2d)


def _sc_a(nt2, aij_cm, nf):
    mesh = plsc.VectorSubcoreMesh(core_axis_name="c", subcore_axis_name="s")

    @functools.partial(
        pl.kernel,
        mesh=mesh,
        compiler_params=pltpu.CompilerParams(needs_layout_passes=False),
        out_type=[
            jax.ShapeDtypeStruct((2 * N, D), jnp.float32),
            jax.ShapeDtypeStruct((8 * NPAD,), jnp.float32),
        ],
        scratch_types=[
            pltpu.VMEM((CH,), jnp.int32),
            pltpu.VMEM((CH,), jnp.int32),
            pltpu.VMEM((CH,), jnp.int32),
            pltpu.VMEM((CH,), jnp.int32),
            pltpu.VMEM((CH,), jnp.int32),
            pltpu.VMEM((CH,), jnp.int32),
            pltpu.VMEM((CH,), jnp.float32),
            pltpu.VMEM((CH,), jnp.float32),
            pltpu.VMEM((CH,), jnp.float32),
            pltpu.VMEM((CH,), jnp.float32),
            pltpu.VMEM((CH, D), jnp.float32),
            pltpu.SemaphoreType.DMA,
        ],
    )
    def k(nt2_h, aij_h, nf_h, xt_o, an_o,
          t1b, gia, gib, gja, gjb, rix_v, ai0b, ai1b, aj0b, aj1b, rows_v,
          sem):
        cid = lax.axis_index("c")
        tid = lax.axis_index("s")
        nbase = cid * N
        abase = cid * 4 * NPAD
        row_lo = tid * 640
        nch0 = jnp.where(tid < 15, 640 // CH, 400 // CH)

        def p0_chunk(c, carry):
            base = row_lo + c * CH
            pltpu.sync_copy(nf_h.at[pl.ds(base, CH)], t1b)
            for g in range(CH // 16):
                o = pl.ds(g * 16, 16)
                v = t1b[o]
                rix_v[o] = v + nbase
                gia[o] = v + abase
                gib[o] = v + (abase + NPAD)
                gja[o] = v + (abase + 2 * NPAD)
                gjb[o] = v + (abase + 3 * NPAD)
            cp_rw = pltpu.async_copy(nt2_h.at[rix_v], rows_v, sem)
            cp_a = pltpu.async_copy(aij_h.at[gia], ai0b, sem)
            cp_b = pltpu.async_copy(aij_h.at[gib], ai1b, sem)
            cp_c = pltpu.async_copy(aij_h.at[gja], aj0b, sem)
            cp_d = pltpu.async_copy(aij_h.at[gjb], aj1b, sem)
            cp_rw.wait()
            cp_a.wait()
            cp_b.wait()
            cp_c.wait()
            cp_d.wait()
            pltpu.sync_copy(rows_v, xt_o.at[pl.ds(nbase + base, CH)])
            pltpu.sync_copy(ai0b, an_o.at[pl.ds(abase + base, CH)])
            pltpu.sync_copy(ai1b, an_o.at[pl.ds(abase + NPAD + base, CH)])
            pltpu.sync_copy(aj0b, an_o.at[pl.ds(abase + 2 * NPAD + base, CH)])
            pltpu.sync_copy(aj1b, an_o.at[pl.ds(abase + 3 * NPAD + base, CH)])
            return carry

        lax.fori_loop(0, nch0, p0_chunk, 0)

    return k(nt2, aij_cm, nf)


def _sc_b(xt, an_cm, eta2, ln_et, ef, src, dst):
    mesh = plsc.VectorSubcoreMesh(core_axis_name="c", subcore_axis_name="s")

    @functools.partial(
        pl.kernel,
        mesh=mesh,
        compiler_params=pltpu.CompilerParams(needs_layout_passes=False),
        out_type=[
            jax.ShapeDtypeStruct((2 * N, D), jnp.float32),
            jax.ShapeDtypeStruct((NPAD,), jnp.float32),
            jax.ShapeDtypeStruct((E, HF), jnp.float32),
        ],
        scratch_types=[
            pltpu.VMEM((4 * EDICT,), jnp.float32),
            pltpu.VMEM((SB,), jnp.int32),
            pltpu.VMEM((SB,), jnp.int32),
            pltpu.VMEM((SB,), jnp.int32),
            pltpu.VMEM((CH,), jnp.int32),
            pltpu.VMEM((CH,), jnp.int32),
            pltpu.VMEM((CH,), jnp.int32),
            pltpu.VMEM((CH,), jnp.int32),
            pltpu.VMEM((CH,), jnp.float32),
            pltpu.VMEM((CH,), jnp.float32),
            pltpu.VMEM((CH,), jnp.float32),
            pltpu.VMEM((CH,), jnp.float32),
            pltpu.VMEM((CH,), jnp.float32),
            pltpu.VMEM((CH,), jnp.float32),
            pltpu.VMEM((CH,), jnp.float32),
            pltpu.VMEM((CH,), jnp.float32),
            pltpu.VMEM((CH,), jnp.int32),
            pltpu.VMEM((CH,), jnp.int32),
            pltpu.VMEM((CH,), jnp.int32),
            pltpu.VMEM((CH,), jnp.int32),
            pltpu.VMEM((CH,), jnp.float32),
            pltpu.VMEM((CH,), jnp.float32),
            pltpu.VMEM((CH,), jnp.float32),
            pltpu.VMEM((CH3,), jnp.int32),
            pltpu.VMEM((CH, D), jnp.float32),
            pltpu.VMEM((CH3, HF), jnp.float32),
            pltpu.VMEM_SHARED((N, D), jnp.float32),
            pltpu.VMEM_SHARED((DEN_PAD,), jnp.float32),
            pltpu.VMEM_SHARED((NPAD,), jnp.float32),
            pltpu.VMEM_SHARED((4 * NPAD,), jnp.float32),
            pltpu.SemaphoreType.DMA,
            pltpu.SemaphoreType.DMA,
        ],
    )
    def k(xt_h, an_h, eta_h, lnet_h, ef_h, src_h, dst_h,
          acc_o, deg_o, eout_o,
          eta_v, s_big, d_big, f_big, gia, gib, gja, gjb,
          ai0b, ai1b, aj0b, aj1b, dn0b, dn1b,
          ex0_v, ex1_v, di0_v, di1_v, dd_v, rix_v, at0_v, at1_v, ones_v,
          fch_v, rows_v, erows_v, accum_sp, den_sp, deg_sp, an_sp, sem,
          sem2):
        cid = lax.axis_index("c")
        tid = lax.axis_index("s")
        nbase = cid * N
        abase = cid * 4 * NPAD
        eb0 = cid * 2 * EDICT

        for g in range(CH // 16):
            ones_v[pl.ds(g * 16, 16)] = jnp.ones((16,), jnp.float32)
            ex0_v[pl.ds(g * 16, 16)] = jnp.zeros((16,), jnp.float32)
        row_lo = tid * 640
        nch0 = jnp.where(tid < 15, 640 // CH, 400 // CH)

        def zrow(r, carry):
            for q in range(D // 16):
                rows_v[r, pl.ds(q * 16, 16)] = jnp.zeros((16,), jnp.float32)
            return carry

        pltpu.sync_copy(eta_h, eta_v)
        for i in range(4 * NPAD // NT // CH):
            lo = tid * (4 * NPAD // NT) + i * CH
            pltpu.sync_copy(an_h.at[pl.ds(abase + lo, CH)], ai0b)
            pltpu.sync_copy(ai0b, an_sp.at[pl.ds(lo, CH)])
        for i in range(DEN_PAD // NT // CH):
            pltpu.sync_copy(
                ex0_v, den_sp.at[pl.ds(tid * (DEN_PAD // NT) + i * CH, CH)])
        for i in range(NPAD // NT // CH):
            pltpu.sync_copy(
                ex0_v, deg_sp.at[pl.ds(tid * (NPAD // NT) + i * CH, CH)])
        lax.fori_loop(0, CH, zrow, 0)

        def zacc(c, carry):
            pltpu.sync_copy(rows_v, accum_sp.at[pl.ds(row_lo + c * CH, CH)])
            return carry

        lax.fori_loop(0, nch0, zacc, 0)

        plsc.subcore_barrier()

        def p1_super(u, carry):
            eb = tid * EPT + u * SB
            pltpu.sync_copy(src_h.at[pl.ds(eb, SB)], s_big)
            pltpu.sync_copy(dst_h.at[pl.ds(eb, SB)], d_big)
            pltpu.sync_copy(ef_h.at[pl.ds(eb, SB)], f_big)

            def p1_chunk(v, c2):
                off = v * CH
                for g in range(CH // 16):
                    o = pl.ds(g * 16, 16)
                    s = s_big[pl.ds(off + g * 16, 16)]
                    d = d_big[pl.ds(off + g * 16, 16)]
                    gia[o] = d
                    gib[o] = d + NPAD
                    gja[o] = s + 2 * NPAD
                    gjb[o] = s + 3 * NPAD
                    di0_v[o] = d * 2
                    di1_v[o] = d * 2 + 1
                    dd_v[o] = d
                cp_a = pltpu.async_copy(an_sp.at[gia], ai0b, sem)
                cp_b = pltpu.async_copy(an_sp.at[gib], ai1b, sem)
                cp_c = pltpu.async_copy(an_sp.at[gja], aj0b, sem)
                cp_d = pltpu.async_copy(an_sp.at[gjb], aj1b, sem)
                cp_a.wait()
                cp_b.wait()
                cp_c.wait()
                cp_d.wait()
                for g in range(CH // 16):
                    o = pl.ds(g * 16, 16)
                    f = f_big[pl.ds(off + g * 16, 16)]
                    a_e0 = plsc.load_gather(eta_v, [eb0 + f * 2])
                    a_e1 = plsc.load_gather(eta_v, [eb0 + f * 2 + 1])
                    l0 = ai0b[o] + aj0b[o] + a_e0
                    l1 = ai1b[o] + aj1b[o] + a_e1
                    l0 = jnp.where(l0 >= 0.0, l0, 0.2 * l0)
                    l1 = jnp.where(l1 >= 0.0, l1, 0.2 * l1)
                    ex0_v[o] = jnp.exp(l0)
                    ex1_v[o] = jnp.exp(l1)
                cs0 = pltpu.async_copy(ex0_v, den_sp.at[di0_v], sem,
                                       add=True)
                cs1 = pltpu.async_copy(ex1_v, den_sp.at[di1_v], sem,
                                       add=True)

                @pl.when(cid == 0)
                def _():
                    pltpu.async_copy(ones_v, deg_sp.at[dd_v], sem,
                                     add=True).wait()
                cs0.wait()
                cs1.wait()
                return c2

            lax.fori_loop(0, SB // CH, p1_chunk, 0)
            return carry

        lax.fori_loop(0, EPT // SB, p1_super, 0)
        plsc.subcore_barrier()

        def p2_super(u, carry):
            eb = tid * EPT + u * SB
            pltpu.sync_copy(src_h.at[pl.ds(eb, SB)], s_big)
            pltpu.sync_copy(dst_h.at[pl.ds(eb, SB)], d_big)
            pltpu.sync_copy(ef_h.at[pl.ds(eb, SB)], f_big)

            def p2_chunk(v, c2):
                off = v * CH
                for g in range(CH // 16):
                    o = pl.ds(g * 16, 16)
                    s = s_big[pl.ds(off + g * 16, 16)]
                    d = d_big[pl.ds(off + g * 16, 16)]
                    gia[o] = d
                    gib[o] = d + NPAD
                    gja[o] = s + 2 * NPAD
                    gjb[o] = s + 3 * NPAD
                    rix_v[o] = s + nbase
                    di0_v[o] = d * 2
                    di1_v[o] = d * 2 + 1
                    dd_v[o] = d
                cp_rw = pltpu.async_copy(xt_h.at[rix_v], rows_v, sem2)
                cp_a = pltpu.async_copy(an_sp.at[gia], ai0b, sem)
                cp_b = pltpu.async_copy(an_sp.at[gib], ai1b, sem)
                cp_c = pltpu.async_copy(an_sp.at[gja], aj0b, sem)
                cp_d = pltpu.async_copy(an_sp.at[gjb], aj1b, sem)
                cp_a.wait()
                cp_b.wait()
                cp_c.wait()
                cp_d.wait()
                cp_d0 = pltpu.async_copy(den_sp.at[di0_v], dn0b, sem)
                cp_d1 = pltpu.async_copy(den_sp.at[di1_v], dn1b, sem)
                cp_d0.wait()
                cp_d1.wait()
                for g in range(CH // 16):
                    o = pl.ds(g * 16, 16)
                    f = f_big[pl.ds(off + g * 16, 16)]
                    a_e0 = plsc.load_gather(eta_v, [eb0 + f * 2])
                    a_e1 = plsc.load_gather(eta_v, [eb0 + f * 2 + 1])
                    l0 = ai0b[o] + aj0b[o] + a_e0
                    l1 = ai1b[o] + aj1b[o] + a_e1
                    l0 = jnp.where(l0 >= 0.0, l0, 0.2 * l0)
                    l1 = jnp.where(l1 >= 0.0, l1, 0.2 * l1)
                    at0_v[o] = jnp.exp(l0) / dn0b[o]
                    at1_v[o] = jnp.exp(l1) / dn1b[o]
                cp_rw.wait()
                for g in range(CH // 16):
                    a0v = at0_v[pl.ds(g * 16, 16)]
                    a1v = at1_v[pl.ds(g * 16, 16)]
                    for b in range(16):
                        r = g * 16 + b
                        ib = jnp.full((16,), b, jnp.int32)
                        c0 = a0v.at[ib].get(mode="promise_in_bounds")
                        c1 = a1v.at[ib].get(mode="promise_in_bounds")
                        for q in range(4):
                            oq = pl.ds(q * 16, 16)
                            rows_v[r, oq] = rows_v[r, oq] * c0
                        for q in range(4, 8):
                            oq = pl.ds(q * 16, 16)
                            rows_v[r, oq] = rows_v[r, oq] * c1
                pltpu.sync_copy(rows_v, accum_sp.at[dd_v], add=True)
                return c2

            lax.fori_loop(0, SB // CH, p2_chunk, 0)
            return carry

        lax.fori_loop(0, EPT // SB, p2_super, 0)
        plsc.subcore_barrier()

        def wacc(c, carry):
            lo = row_lo + c * CH
            pltpu.sync_copy(accum_sp.at[pl.ds(lo, CH)], rows_v)
            pltpu.sync_copy(rows_v, acc_o.at[pl.ds(nbase + lo, CH)])
            return carry

        lax.fori_loop(0, nch0, wacc, 0)

        @pl.when(cid == 0)
        def _():
            for i in range(NPAD // NT // CH):
                lo = tid * (NPAD // NT) + i * CH
                pltpu.sync_copy(deg_sp.at[pl.ds(lo, CH)], ex0_v)
                pltpu.sync_copy(ex0_v, deg_o.at[pl.ds(lo, CH)])

        def p3_chunk(c, carry):
            base = cid * (E // 2) + tid * EPT3 + c * CH3
            pltpu.sync_copy(ef_h.at[pl.ds(base, CH3)], fch_v)
            pltpu.async_copy(lnet_h.at[fch_v], erows_v, sem).wait()
            pltpu.sync_copy(erows_v, eout_o.at[pl.ds(base, CH3)])
            return carry

        lax.fori_loop(0, EPT3 // CH3, p3_chunk, 0)

    return k(xt, an_cm, eta2, ln_et, ef, src, dst)


def kernel(node_table, edge_table, Wt, bt, We, be, Wa, ba, gamma, beta,
           node_features, edge_features, edge_index):
    f32 = jnp.float32
    nf = node_features.astype(jnp.int32)
    ef = edge_features.astype(jnp.int32)
    src = edge_index[0].astype(jnp.int32)
    dst = edge_index[1].astype(jnp.int32)
    wa1 = Wa[0:F, 0]
    wa2 = Wa[F:2 * F, 0]
    wa3 = Wa[2 * F:3 * F, 0]
    eye = jnp.eye(H, dtype=f32)
    w1 = jnp.kron(eye, wa1[:, None])
    w2 = jnp.kron(eye, wa2[:, None])
    w12 = jnp.concatenate([w1, w2], axis=1)
    w3 = jnp.concatenate([jnp.kron(eye, wa3[:, None]),
                          jnp.zeros((HF, H), f32)], axis=1)
    nt2, aiaj, eta8, ln_et = _tc1(
        node_table.astype(f32), edge_table.astype(f32), Wt.astype(f32),
        bt.reshape(1, HF).astype(f32), We.astype(f32),
        be.reshape(1, HF).astype(f32), w12, w3,
        ba.reshape(1, 1).astype(f32), gamma.reshape(1, HF).astype(f32),
        beta.reshape(1, HF).astype(f32))
    ai = aiaj[:, 0:H]
    aj = aiaj[:, H:2 * H]
    et_a = eta8[:, 0:H]
    pad = jnp.zeros((NPAD - N,), f32)
    cols = []
    for core in range(2):
        for col in (ai[:, 2 * core], ai[:, 2 * core + 1],
                    aj[:, 2 * core], aj[:, 2 * core + 1]):
            cols.append(jnp.concatenate([col, pad]))
    aij_cm = jnp.concatenate(cols)
    eta2 = jnp.concatenate([et_a[:, 0:2].reshape(-1),
                            et_a[:, 2:4].reshape(-1)])
    xt, an_cm = _sc_a(nt2, aij_cm, nf)
    acc, deg, edge_out = _sc_b(xt, an_cm, eta2, ln_et, ef, src, dst)
    deg2d = deg[0:N].reshape(N, 1)
    out = _tc2(acc, xt, deg2d, gamma.reshape(1, HF).astype(f32),
               beta.reshape(1, HF).astype(f32))
    return (out, edge_out)
